# Initial kernel scaffold; baseline (speedup 1.0000x reference)
import jax
import jax.numpy as jnp
from jax.experimental import pallas as pl


def kernel(image_feature, labevent_number_input, labevent_category_input, microbiology_category_input, microbiology_number_input, microbiology_comment_embeddings, medical_history_embeddings, family_history_embeddings, patient_category_input, patient_number_input, triage_category_input, triage_number_input, chiefcomplaint_embedding, total_attention_mask, multimodal_input_type, labevent_table, spec_table, test_table, org_table, ab_table, dil_table, patient_table, triage_table, W_lab, b_lab, W_micro, b_micro, W_age, b_age, W_triage, b_triage):
    raise NotImplementedError("write your pallas kernel here")


# speedup vs baseline: 1.8247x; 1.8247x over previous
import jax
import jax.numpy as jnp
from jax.experimental import pallas as pl

B = 32
H = 768
NB = 2000


def _onehot_gather(table, idx, n):
    oh = (jax.lax.broadcasted_iota(jnp.int32, (idx.shape[0], n), 1)
          == idx[:, None]).astype(jnp.float32)
    return jnp.dot(oh, table, preferred_element_type=jnp.float32)


def _body(img_ref, lab_num_ref, lab_idx_ref,
          spec_idx_ref, test_idx_ref, org_idx_ref, ab_idx_ref, dil_idx_ref,
          micro_num_ref, micro_com_ref, med_ref, fam_ref,
          pat_idx_ref, pat_num_ref, tri_idx_ref, tri_num_ref, chief_ref,
          lab_tab_ref, spec_tab_ref, test_tab_ref, org_tab_ref, ab_tab_ref,
          dil_tab_ref, pat_tab_ref, tri_tab_ref,
          Wlab_ref, blab_ref, Wmic_ref, bmic_ref,
          Wage_ref, bage_ref, Wtri_ref, btri_ref,
          out_ref):
    relu = lambda x: jnp.maximum(x, 0.0)

    out_ref[0, 0:256] = img_ref[0]

    val = relu(jnp.dot(lab_num_ref[0], Wlab_ref[:],
                       preferred_element_type=jnp.float32) + blab_ref[:])
    ids = _onehot_gather(lab_tab_ref[:], lab_idx_ref[0, 0], 1000)
    lab = jnp.concatenate([val, ids], axis=1).reshape(400, H)
    out_ref[0, 256:656] = lab

    spec_f = _onehot_gather(spec_tab_ref[:], spec_idx_ref[0, 0], 200)
    test_f = _onehot_gather(test_tab_ref[:], test_idx_ref[0, 0], 200)
    org_f = _onehot_gather(org_tab_ref[:], org_idx_ref[0, 0], 800)
    ab_f = _onehot_gather(ab_tab_ref[:], ab_idx_ref[0, 0], 100)
    dil_f = _onehot_gather(dil_tab_ref[:], dil_idx_ref[0, 0], 16)
    dil_val = relu(jnp.dot(micro_num_ref[0], Wmic_ref[:],
                           preferred_element_type=jnp.float32) + bmic_ref[:])
    micro = jnp.concatenate(
        [spec_f, test_f, org_f, ab_f, dil_f, dil_val, micro_com_ref[0]],
        axis=1).reshape(105, H)
    out_ref[0, 656:761] = micro

    out_ref[0, 761:889] = med_ref[0]
    out_ref[0, 889:953] = fam_ref[0]

    pat_f = _onehot_gather(pat_tab_ref[:], pat_idx_ref[0, 0], 64)
    age = relu(jnp.dot(pat_num_ref[0], Wage_ref[:],
                       preferred_element_type=jnp.float32) + bage_ref[:])
    out_ref[0, 953:956] = pat_f
    out_ref[0, 956:957] = age

    vit = relu(jnp.dot(tri_num_ref[0], Wtri_ref[:],
                       preferred_element_type=jnp.float32) + btri_ref[:])
    pa_f = _onehot_gather(tri_tab_ref[:], tri_idx_ref[0, 0], 32)
    out_ref[0, 957:963] = vit
    out_ref[0, 963:965] = pa_f

    out_ref[0, 965:997] = chief_ref[0]


def kernel(image_feature, labevent_number_input, labevent_category_input,
           microbiology_category_input, microbiology_number_input,
           microbiology_comment_embeddings, medical_history_embeddings,
           family_history_embeddings, patient_category_input,
           patient_number_input, triage_category_input, triage_number_input,
           chiefcomplaint_embedding, total_attention_mask,
           multimodal_input_type, labevent_table, spec_table, test_table,
           org_table, ab_table, dil_table, patient_table, triage_table,
           W_lab, b_lab, W_micro, b_micro, W_age, b_age, W_triage, b_triage):
    i32 = jnp.int32
    lab_idx = labevent_category_input.astype(i32).reshape(B, 1, 200)
    spec_idx = microbiology_category_input[:, 0::5].astype(i32).reshape(B, 1, 15)
    test_idx = microbiology_category_input[:, 1::5].astype(i32).reshape(B, 1, 15)
    org_idx = microbiology_category_input[:, 2::5].astype(i32).reshape(B, 1, 15)
    ab_idx = microbiology_category_input[:, 3::5].astype(i32).reshape(B, 1, 15)
    dil_idx = microbiology_category_input[:, 4::5].astype(i32).reshape(B, 1, 15)
    pat_idx = patient_category_input.astype(i32).reshape(B, 1, 3)
    tri_idx = triage_category_input[:, -2:].astype(i32).reshape(B, 1, 2)

    def batch_spec(shape):
        nd = len(shape)
        return pl.BlockSpec((1,) + shape,
                            lambda b: (b,) + (0,) * nd)

    def const_spec(shape):
        return pl.BlockSpec(shape, lambda b: (0,) * len(shape))

    in_specs = [
        batch_spec((256, H)),
        batch_spec((200, NB)),
        batch_spec((1, 200)),
        batch_spec((1, 15)),
        batch_spec((1, 15)),
        batch_spec((1, 15)),
        batch_spec((1, 15)),
        batch_spec((1, 15)),
        batch_spec((15, NB)),
        batch_spec((15, H)),
        batch_spec((128, H)),
        batch_spec((64, H)),
        batch_spec((1, 3)),
        batch_spec((1, NB)),
        batch_spec((1, 2)),
        batch_spec((6, NB)),
        batch_spec((32, H)),
        const_spec((1000, H)),
        const_spec((200, H)),
        const_spec((200, H)),
        const_spec((800, H)),
        const_spec((100, H)),
        const_spec((16, H)),
        const_spec((64, H)),
        const_spec((32, H)),
        const_spec((NB, H)),
        const_spec((1, H)),
        const_spec((NB, H)),
        const_spec((1, H)),
        const_spec((NB, H)),
        const_spec((1, H)),
        const_spec((NB, H)),
        const_spec((1, H)),
    ]

    out = pl.pallas_call(
        _body,
        grid=(B,),
        in_specs=in_specs,
        out_specs=pl.BlockSpec((1, 997, H), lambda b: (b, 0, 0)),
        out_shape=jax.ShapeDtypeStruct((B, 997, H), jnp.float32),
    )(
        image_feature, labevent_number_input, lab_idx,
        spec_idx, test_idx, org_idx, ab_idx, dil_idx,
        microbiology_number_input, microbiology_comment_embeddings,
        medical_history_embeddings, family_history_embeddings,
        pat_idx, patient_number_input.astype(jnp.float32),
        tri_idx, triage_number_input.astype(jnp.float32),
        chiefcomplaint_embedding,
        labevent_table, spec_table, test_table, org_table, ab_table,
        dil_table, patient_table, triage_table,
        W_lab, b_lab.reshape(1, H), W_micro, b_micro.reshape(1, H),
        W_age, b_age.reshape(1, H), W_triage, b_triage.reshape(1, H),
    )
    return out
